# baseline (speedup 1.0000x reference)
import jax
import jax.numpy as jnp
from jax import lax
from jax.experimental import pallas as pl
from jax.experimental.pallas import tpu as pltpu
from jax.experimental.pallas import tpu_sc as plsc

VOCAB = 50257
SEQ_LEN = 2048
HIDDEN = 1024
BATCH = 4

NC = 2
NS = 16
LANES = 16
NW = NC * NS

TOKENS = BATCH * SEQ_LEN
POSW = SEQ_LEN // NW
CHUNK = 8
NBLK = POSW // CHUNK
VPR = HIDDEN // LANES
DEPTH = 3
NBUF = DEPTH * BATCH


def _body(ids_hbm, wte_hbm, wpe_hbm, out_hbm, idx_v, pa, pb, *rest):
    bufs = rest[:NBUF]
    isem, psem = rest[NBUF], rest[NBUF + 1]
    gsems = rest[NBUF + 2:2 * NBUF + 2]
    ssems = rest[2 * NBUF + 2:]
    posb = (pa, pb)

    wid = lax.axis_index("s") * NC + lax.axis_index("c")
    p0 = wid * POSW

    def g_src(p, b):
        return wte_hbm.at[idx_v.at[b, pl.ds(p * CHUNK, CHUNK)]]

    def out_dst(p, b):
        return out_hbm.at[pl.ds(b * SEQ_LEN + p0 + p * CHUNK, CHUNK)]

    def pos_src(p):
        return wpe_hbm.at[pl.ds(p0 + p * CHUNK, CHUNK)]

    for b in range(BATCH):
        pltpu.async_copy(ids_hbm.at[pl.ds(b * SEQ_LEN + p0, POSW)],
                         idx_v.at[b], isem)
    pltpu.async_copy(pos_src(0), pa, psem)
    for b in range(BATCH):
        pltpu.make_async_copy(ids_hbm.at[pl.ds(b * SEQ_LEN + p0, POSW)],
                              idx_v.at[b], isem).wait()
    for p in range(DEPTH):
        for b in range(BATCH):
            k = (p % DEPTH) * BATCH + b
            pltpu.async_copy(g_src(p, b), bufs[k], gsems[k])

    for p in range(NBLK):
        pos_v = posb[p % 2]
        pltpu.make_async_copy(pos_src(p), pos_v, psem).wait()
        if p < NBLK - 1:
            pltpu.async_copy(pos_src(p + 1), posb[(p + 1) % 2], psem)
        for b in range(BATCH):
            k = (p % DEPTH) * BATCH + b
            pltpu.make_async_copy(g_src(p, b), bufs[k], gsems[k]).wait()

            @plsc.parallel_loop(0, CHUNK * VPR, 1, unroll=8)
            def _add_v(v, k=k, pos_v=pos_v):
                r = v >> 6
                jcol = (v & (VPR - 1)) * LANES
                plsc.addupdate(bufs[k].at[r, pl.ds(jcol, LANES)],
                               pos_v[r, pl.ds(jcol, LANES)])

            pltpu.async_copy(bufs[k], out_dst(p, b), ssems[k])
        for b in range(BATCH):
            k = (p % DEPTH) * BATCH + b
            pltpu.make_async_copy(bufs[k], out_dst(p, b), ssems[k]).wait()
            if p < NBLK - DEPTH:
                pltpu.async_copy(g_src(p + DEPTH, b), bufs[k], gsems[k])


@jax.jit
def _embed(ids, wte, wpe):
    mesh = plsc.VectorSubcoreMesh(core_axis_name="c", subcore_axis_name="s")
    return pl.kernel(
        _body,
        out_type=jax.ShapeDtypeStruct((TOKENS, HIDDEN), jnp.float32),
        mesh=mesh,
        scratch_types=[
            pltpu.VMEM((BATCH, POSW), jnp.int32),
            pltpu.VMEM((CHUNK, HIDDEN), jnp.float32),
            pltpu.VMEM((CHUNK, HIDDEN), jnp.float32),
        ] + [pltpu.VMEM((CHUNK, HIDDEN), jnp.float32)] * NBUF
          + [pltpu.SemaphoreType.DMA] * (2 + 2 * NBUF),
    )(ids, wte, wpe)


def kernel(input_ids, token_embeddings, position_embeddings):
    ids = input_ids.reshape(-1).astype(jnp.int32)
    out = _embed(ids, token_embeddings, position_embeddings)
    return out.reshape(BATCH, SEQ_LEN, HIDDEN)


# speedup vs baseline: 1.0421x; 1.0421x over previous
import jax
import jax.numpy as jnp
from jax import lax
from jax.experimental import pallas as pl
from jax.experimental.pallas import tpu as pltpu
from jax.experimental.pallas import tpu_sc as plsc

VOCAB = 50257
SEQ_LEN = 2048
HIDDEN = 1024
BATCH = 4

NC = 2
NS = 16
LANES = 16
NW = NC * NS

TOKENS = BATCH * SEQ_LEN
POSW = SEQ_LEN // NW
CHUNK = 8
NBLK = POSW // CHUNK
VPR = HIDDEN // LANES
DEPTH = 2
NBUF = DEPTH * BATCH


def _body(ids_hbm, wte_hbm, wpe_hbm, out_hbm, idx_v, pa, pb, *rest):
    bufs = rest[:NBUF]
    isem, psem = rest[NBUF], rest[NBUF + 1]
    gsems = rest[NBUF + 2:2 * NBUF + 2]
    ssems = rest[2 * NBUF + 2:]
    posb = (pa, pb)

    wid = lax.axis_index("s") * NC + lax.axis_index("c")
    p0 = wid * POSW

    def g_src(p, b):
        return wte_hbm.at[idx_v.at[b, pl.ds(p * CHUNK, CHUNK)]]

    def out_dst(p, b):
        return out_hbm.at[pl.ds(b * SEQ_LEN + p0 + p * CHUNK, CHUNK)]

    def pos_src(p):
        return wpe_hbm.at[pl.ds(p0 + p * CHUNK, CHUNK)]

    for b in range(BATCH):
        pltpu.async_copy(ids_hbm.at[pl.ds(b * SEQ_LEN + p0, POSW)],
                         idx_v.at[b], isem)
    pltpu.async_copy(pos_src(0), pa, psem)
    for b in range(BATCH):
        pltpu.make_async_copy(ids_hbm.at[pl.ds(b * SEQ_LEN + p0, POSW)],
                              idx_v.at[b], isem).wait()
    for p in range(DEPTH):
        for b in range(BATCH):
            k = (p % DEPTH) * BATCH + b
            pltpu.async_copy(g_src(p, b), bufs[k], gsems[k])

    for p in range(NBLK):
        pos_v = posb[p % 2]
        pltpu.make_async_copy(pos_src(p), pos_v, psem).wait()
        if p < NBLK - 1:
            pltpu.async_copy(pos_src(p + 1), posb[(p + 1) % 2], psem)
        for b in range(BATCH):
            k = (p % DEPTH) * BATCH + b
            pltpu.make_async_copy(g_src(p, b), bufs[k], gsems[k]).wait()

            @plsc.parallel_loop(0, CHUNK * VPR, 1, unroll=4)
            def _add_v(v, k=k, pos_v=pos_v):
                r = v >> 6
                jcol = (v & (VPR - 1)) * LANES
                plsc.addupdate(bufs[k].at[r, pl.ds(jcol, LANES)],
                               pos_v[r, pl.ds(jcol, LANES)])

            pltpu.async_copy(bufs[k], out_dst(p, b), ssems[k])
        for b in range(BATCH):
            k = (p % DEPTH) * BATCH + b
            pltpu.make_async_copy(bufs[k], out_dst(p, b), ssems[k]).wait()
            if p < NBLK - DEPTH:
                pltpu.async_copy(g_src(p + DEPTH, b), bufs[k], gsems[k])


@jax.jit
def _embed(ids, wte, wpe):
    mesh = plsc.VectorSubcoreMesh(core_axis_name="c", subcore_axis_name="s")
    return pl.kernel(
        _body,
        out_type=jax.ShapeDtypeStruct((TOKENS, HIDDEN), jnp.float32),
        mesh=mesh,
        scratch_types=[
            pltpu.VMEM((BATCH, POSW), jnp.int32),
            pltpu.VMEM((CHUNK, HIDDEN), jnp.float32),
            pltpu.VMEM((CHUNK, HIDDEN), jnp.float32),
        ] + [pltpu.VMEM((CHUNK, HIDDEN), jnp.float32)] * NBUF
          + [pltpu.SemaphoreType.DMA] * (2 + 2 * NBUF),
    )(ids, wte, wpe)


def kernel(input_ids, token_embeddings, position_embeddings):
    ids = input_ids.reshape(-1).astype(jnp.int32)
    out = _embed(ids, token_embeddings, position_embeddings)
    return out.reshape(BATCH, SEQ_LEN, HIDDEN)
